# baseline (speedup 1.0000x reference)
import functools

import jax
import jax.numpy as jnp
from jax import lax
from jax.experimental import pallas as pl
from jax.experimental.pallas import tpu as pltpu
from jax.experimental.pallas import tpu_sc as plsc

N = 10000
D = 128
E = 320000

NC = 2
NS = 16
NW = NC * NS
EPB = 128
NBUF = 4
NB_PER_W = 80
E_PAD = NB_PER_W * NW * EPB
PAD = E_PAD - E
ACC_ROWS = 10240
TRASH = N
RPS = ACC_ROWS // NS
ZROWS = 64
DEGW = 128
NBT = E_PAD // EPB
NB0 = 64
NB1 = NBT // NS - NB0

BR = 1000

_MESH = dict(core_axis_name="c", subcore_axis_name="s")


def _sc_degree(dp):
    @functools.partial(
        pl.kernel,
        out_type=jax.ShapeDtypeStruct((NC, ACC_ROWS, DEGW), jnp.float32),
        mesh=plsc.VectorSubcoreMesh(**_MESH),
        scratch_types=[pltpu.VMEM((EPB, DEGW), jnp.float32)]
        + [pltpu.VMEM((EPB,), jnp.int32)] * 4
        + [pltpu.VMEM_SHARED((ACC_ROWS, DEGW), jnp.float32)]
        + [pltpu.SemaphoreType.DMA] * 8,
    )
    def k(dp_hbm, out_hbm, *bufs):
        ones = bufs[0]
        didx = bufs[1:5]
        acc = bufs[5]
        isem, ssem = bufs[6:10], bufs[10:14]
        cid = lax.axis_index("c")
        sid = lax.axis_index("s")
        gwid = cid * NS + sid

        def issue_idx(b, ji):
            pltpu.async_copy(dp_hbm.at[gwid, b], didx[ji], isem[ji])

        def wait_idx(b, ji):
            pltpu.make_async_copy(dp_hbm.at[gwid, b], didx[ji],
                                  isem[ji]).wait()

        def issue_s(ji):
            pltpu.async_copy(ones, acc.at[didx[ji]], ssem[ji], add=True)

        def wait_s(ji):
            pltpu.make_async_copy(ones, acc.at[didx[ji]], ssem[ji]).wait()

        issue_idx(0, 0)

        @pl.loop(0, EPB)
        def _(r):
            @pl.loop(0, DEGW // 16)
            def _(c):
                ones[r, pl.ds(c * 16, 16)] = jnp.ones((16,), jnp.float32)

        @pl.loop(0, RPS // ZROWS)
        def _(i):
            pltpu.sync_copy(ones.at[pl.ds(0, ZROWS)],
                            acc.at[pl.ds(sid * RPS + i * ZROWS, ZROWS)])

        plsc.subcore_barrier()

        @pl.loop(0, NB_PER_W // 4)
        def _(g):
            for j in range(4):
                b = g * 4 + j
                ji1, ji3 = (j + 1) % 4, (j + 3) % 4

                @pl.when(b >= 3)
                def _():
                    wait_s(ji1)

                @pl.when(b + 1 < NB_PER_W)
                def _():
                    issue_idx(b + 1, ji1)

                wait_idx(b, j)
                issue_s(j)

        for j in (1, 2, 3):
            wait_s(j)

        plsc.subcore_barrier()

        @pl.loop(0, RPS // ZROWS)
        def _(i):
            off = sid * RPS + i * ZROWS
            pltpu.sync_copy(acc.at[pl.ds(off, ZROWS)],
                            out_hbm.at[cid, pl.ds(off, ZROWS)])

    return k(dp)


def _sc_propagate(table, sp, dp):
    @functools.partial(
        pl.kernel,
        out_type=jax.ShapeDtypeStruct((NC, ACC_ROWS, D), jnp.float32),
        mesh=plsc.VectorSubcoreMesh(**_MESH),
        scratch_types=[pltpu.VMEM((EPB, D), jnp.float32)] * 2
        + [pltpu.VMEM((EPB,), jnp.int32)] * 8
        + [pltpu.VMEM_SHARED((ACC_ROWS, D), jnp.float32)]
        + [pltpu.SemaphoreType.DMA] * 8,
    )
    def k(tab_hbm, sp_hbm, dp_hbm, out_hbm, *bufs):
        rows = bufs[0:2]
        sidx = bufs[2:6]
        didx = bufs[6:10]
        acc = bufs[10]
        gsem, ssem = bufs[11:13], bufs[13:15]
        isem = bufs[15:19]
        cid = lax.axis_index("c")
        sid = lax.axis_index("s")
        nb = jnp.where(cid == 0, NB0, NB1)
        base = jnp.where(cid == 0, sid * NB0, NS * NB0 + sid * NB1)

        def issue_idx(b, ji):
            pltpu.async_copy(sp_hbm.at[base + b], sidx[ji], isem[ji])
            pltpu.async_copy(dp_hbm.at[base + b], didx[ji], isem[ji])

        def wait_idx(b, ji):
            pltpu.make_async_copy(sp_hbm.at[base + b], sidx[ji],
                                  isem[ji]).wait()
            pltpu.make_async_copy(dp_hbm.at[base + b], didx[ji],
                                  isem[ji]).wait()

        def issue_g(jr, ji):
            pltpu.async_copy(tab_hbm.at[sidx[ji]], rows[jr], gsem[jr])

        def wait_g(jr, ji):
            pltpu.make_async_copy(tab_hbm.at[sidx[ji]], rows[jr],
                                  gsem[jr]).wait()

        def issue_s(jr, ji):
            pltpu.async_copy(rows[jr], acc.at[didx[ji]], ssem[jr], add=True)

        def wait_s(jr, ji):
            pltpu.make_async_copy(rows[jr], acc.at[didx[ji]],
                                  ssem[jr]).wait()

        for j in range(4):
            issue_idx(j, j)

        @pl.loop(0, EPB)
        def _(r):
            @pl.loop(0, D // 16)
            def _(c):
                rows[0][r, pl.ds(c * 16, 16)] = jnp.zeros((16,), jnp.float32)

        @pl.loop(0, RPS // EPB)
        def _(i):
            pltpu.sync_copy(rows[0], acc.at[pl.ds(sid * RPS + i * EPB, EPB)])

        plsc.subcore_barrier()
        wait_idx(0, 0)
        issue_g(0, 0)

        @pl.loop(0, nb // 4)
        def _(g):
            for j in range(4):
                b = g * 4 + j
                jr, jr1 = j % 2, (j + 1) % 2
                ji, ji1, ji3 = j, (j + 1) % 4, (j + 3) % 4

                @pl.when(b >= 1)
                def _():
                    wait_s(jr1, ji3)

                    @pl.when(b + 3 < nb)
                    def _():
                        issue_idx(b + 3, ji3)

                @pl.when(b + 1 < nb)
                def _():
                    wait_idx(b + 1, ji1)
                    issue_g(jr1, ji1)

                wait_g(jr, ji)
                issue_s(jr, ji)

        wait_s(1, 3)
        plsc.subcore_barrier()

        @pl.loop(0, RPS // EPB)
        def _(i):
            off = sid * RPS + i * EPB
            pltpu.sync_copy(acc.at[pl.ds(off, EPB)],
                            out_hbm.at[cid, pl.ds(off, EPB)])

    return k(table, sp, dp)


def _tc_pre(accd, x):
    def body(accd_ref, x_ref, xs_ref, d_ref):
        deg = accd_ref[0, :, 0:1] + accd_ref[1, :, 0:1] - 1.0
        d = lax.rsqrt(deg)
        d_ref[...] = d
        xs_ref[...] = x_ref[...] * d

    return pl.pallas_call(
        body,
        grid=(N // BR,),
        in_specs=[
            pl.BlockSpec((NC, BR, DEGW), lambda i: (0, i, 0)),
            pl.BlockSpec((BR, D), lambda i: (i, 0)),
        ],
        out_specs=[
            pl.BlockSpec((BR, D), lambda i: (i, 0)),
            pl.BlockSpec((BR, 1), lambda i: (i, 0)),
        ],
        out_shape=[
            jax.ShapeDtypeStruct((N, D), jnp.float32),
            jax.ShapeDtypeStruct((N, 1), jnp.float32),
        ],
    )(accd, x)


def _tc_mid(acc, xs, d, W1, b1):
    def body(acc_ref, xs_ref, d_ref, w_ref, b_ref, emb_ref, es_ref):
        d = d_ref[...]
        h = (acc_ref[0] + acc_ref[1] + xs_ref[...]) * d
        e = lax.dot_general(h, w_ref[...],
                            dimension_numbers=(((1,), (1,)), ((), ())),
                            preferred_element_type=jnp.float32)
        e = jnp.maximum(e + b_ref[...], 0.0)
        emb_ref[...] = e
        es_ref[...] = e * d

    return pl.pallas_call(
        body,
        grid=(N // BR,),
        in_specs=[
            pl.BlockSpec((NC, BR, D), lambda i: (0, i, 0)),
            pl.BlockSpec((BR, D), lambda i: (i, 0)),
            pl.BlockSpec((BR, 1), lambda i: (i, 0)),
            pl.BlockSpec((D, D), lambda i: (0, 0)),
            pl.BlockSpec((1, D), lambda i: (0, 0)),
        ],
        out_specs=[
            pl.BlockSpec((BR, D), lambda i: (i, 0)),
            pl.BlockSpec((BR, D), lambda i: (i, 0)),
        ],
        out_shape=[
            jax.ShapeDtypeStruct((N, D), jnp.float32),
            jax.ShapeDtypeStruct((N, D), jnp.float32),
        ],
    )(acc, xs, d, W1, b1)


def _tc_post(acc, es, d, W2, b2):
    def body(acc_ref, es_ref, d_ref, w_ref, b_ref, out_ref):
        h = (acc_ref[0] + acc_ref[1] + es_ref[...]) * d_ref[...]
        o = lax.dot_general(h, w_ref[...],
                            dimension_numbers=(((1,), (1,)), ((), ())),
                            preferred_element_type=jnp.float32)
        out_ref[...] = o + b_ref[...]

    return pl.pallas_call(
        body,
        grid=(N // BR,),
        in_specs=[
            pl.BlockSpec((NC, BR, D), lambda i: (0, i, 0)),
            pl.BlockSpec((BR, D), lambda i: (i, 0)),
            pl.BlockSpec((BR, 1), lambda i: (i, 0)),
            pl.BlockSpec((D, D), lambda i: (0, 0)),
            pl.BlockSpec((1, D), lambda i: (0, 0)),
        ],
        out_specs=pl.BlockSpec((BR, D), lambda i: (i, 0)),
        out_shape=jax.ShapeDtypeStruct((N, D), jnp.float32),
    )(acc, es, d, W2, b2)


def kernel(x, edge_index, W1, b1, W2, b2):
    sp = jnp.concatenate([edge_index[0], jnp.zeros((PAD,), jnp.int32)])
    dp = jnp.concatenate([edge_index[1], jnp.full((PAD,), TRASH, jnp.int32)])
    spd = sp.reshape(NW, NB_PER_W, EPB)
    dpd = dp.reshape(NW, NB_PER_W, EPB)
    spp = sp.reshape(NBT, EPB)
    dpp = dp.reshape(NBT, EPB)
    accd = _sc_degree(dpd)
    xs, d = _tc_pre(accd, x)
    acc1 = _sc_propagate(xs, spp, dpp)
    emb, es = _tc_mid(acc1, xs, d, W1, b1.reshape(1, D))
    acc2 = _sc_propagate(es, spp, dpp)
    out = _tc_post(acc2, es, d, W2, b2.reshape(1, D))
    return emb, out


# speedup vs baseline: 1.0889x; 1.0889x over previous
import functools

import jax
import jax.numpy as jnp
from jax import lax
from jax.experimental import pallas as pl
from jax.experimental.pallas import tpu as pltpu
from jax.experimental.pallas import tpu_sc as plsc

N = 10000
D = 128
E = 320000

NC = 2
NS = 16
NW = NC * NS
EPB = 128
NBUF = 4
NB_PER_W = 80
E_PAD = NB_PER_W * NW * EPB
PAD = E_PAD - E
ACC_ROWS = 10240
TRASH = N
RPS = ACC_ROWS // NS
ZROWS = 64
DEGW = 128
NBT = E_PAD // EPB
NB0 = 80
NB1 = NBT // NS - NB0

BR = 1000

_MESH = dict(core_axis_name="c", subcore_axis_name="s")


def _sc_degree(dp):
    @functools.partial(
        pl.kernel,
        out_type=jax.ShapeDtypeStruct((NC, ACC_ROWS, DEGW), jnp.float32),
        mesh=plsc.VectorSubcoreMesh(**_MESH),
        scratch_types=[pltpu.VMEM((EPB, DEGW), jnp.float32)]
        + [pltpu.VMEM((EPB,), jnp.int32)] * 4
        + [pltpu.VMEM_SHARED((ACC_ROWS, DEGW), jnp.float32)]
        + [pltpu.SemaphoreType.DMA] * 8,
    )
    def k(dp_hbm, out_hbm, *bufs):
        ones = bufs[0]
        didx = bufs[1:5]
        acc = bufs[5]
        isem, ssem = bufs[6:10], bufs[10:14]
        cid = lax.axis_index("c")
        sid = lax.axis_index("s")
        gwid = cid * NS + sid

        def issue_idx(b, ji):
            pltpu.async_copy(dp_hbm.at[gwid, b], didx[ji], isem[ji])

        def wait_idx(b, ji):
            pltpu.make_async_copy(dp_hbm.at[gwid, b], didx[ji],
                                  isem[ji]).wait()

        def issue_s(ji):
            pltpu.async_copy(ones, acc.at[didx[ji]], ssem[ji], add=True)

        def wait_s(ji):
            pltpu.make_async_copy(ones, acc.at[didx[ji]], ssem[ji]).wait()

        issue_idx(0, 0)

        @pl.loop(0, EPB)
        def _(r):
            @pl.loop(0, DEGW // 16)
            def _(c):
                ones[r, pl.ds(c * 16, 16)] = jnp.ones((16,), jnp.float32)

        @pl.loop(0, RPS // ZROWS)
        def _(i):
            pltpu.sync_copy(ones.at[pl.ds(0, ZROWS)],
                            acc.at[pl.ds(sid * RPS + i * ZROWS, ZROWS)])

        plsc.subcore_barrier()

        @pl.loop(0, NB_PER_W // 4)
        def _(g):
            for j in range(4):
                b = g * 4 + j
                ji1, ji3 = (j + 1) % 4, (j + 3) % 4

                @pl.when(b >= 3)
                def _():
                    wait_s(ji1)

                @pl.when(b + 1 < NB_PER_W)
                def _():
                    issue_idx(b + 1, ji1)

                wait_idx(b, j)
                issue_s(j)

        for j in (1, 2, 3):
            wait_s(j)

        plsc.subcore_barrier()

        @pl.loop(0, RPS // ZROWS)
        def _(i):
            off = sid * RPS + i * ZROWS
            pltpu.sync_copy(acc.at[pl.ds(off, ZROWS)],
                            out_hbm.at[cid, pl.ds(off, ZROWS)])

    return k(dp)


def _sc_propagate(table, sp, dp):
    @functools.partial(
        pl.kernel,
        out_type=jax.ShapeDtypeStruct((NC, ACC_ROWS, D), jnp.float32),
        mesh=plsc.VectorSubcoreMesh(**_MESH),
        scratch_types=[pltpu.VMEM((EPB, D), jnp.float32)] * 2
        + [pltpu.VMEM((EPB,), jnp.int32)] * 8
        + [pltpu.VMEM_SHARED((ACC_ROWS, D), jnp.float32)]
        + [pltpu.SemaphoreType.DMA] * 8,
    )
    def k(tab_hbm, sp_hbm, dp_hbm, out_hbm, *bufs):
        rows = bufs[0:2]
        sidx = bufs[2:6]
        didx = bufs[6:10]
        acc = bufs[10]
        gsem, ssem = bufs[11:13], bufs[13:15]
        isem = bufs[15:19]
        cid = lax.axis_index("c")
        sid = lax.axis_index("s")
        nb = jnp.where(cid == 0, NB0, NB1)
        base = jnp.where(cid == 0, sid * NB0, NS * NB0 + sid * NB1)

        def issue_idx(b, ji):
            pltpu.async_copy(sp_hbm.at[base + b], sidx[ji], isem[ji])
            pltpu.async_copy(dp_hbm.at[base + b], didx[ji], isem[ji])

        def wait_idx(b, ji):
            pltpu.make_async_copy(sp_hbm.at[base + b], sidx[ji],
                                  isem[ji]).wait()
            pltpu.make_async_copy(dp_hbm.at[base + b], didx[ji],
                                  isem[ji]).wait()

        def issue_g(jr, ji):
            pltpu.async_copy(tab_hbm.at[sidx[ji]], rows[jr], gsem[jr])

        def wait_g(jr, ji):
            pltpu.make_async_copy(tab_hbm.at[sidx[ji]], rows[jr],
                                  gsem[jr]).wait()

        def issue_s(jr, ji):
            pltpu.async_copy(rows[jr], acc.at[didx[ji]], ssem[jr], add=True)

        def wait_s(jr, ji):
            pltpu.make_async_copy(rows[jr], acc.at[didx[ji]],
                                  ssem[jr]).wait()

        for j in range(4):
            issue_idx(j, j)

        @pl.loop(0, EPB)
        def _(r):
            @pl.loop(0, D // 16)
            def _(c):
                rows[0][r, pl.ds(c * 16, 16)] = jnp.zeros((16,), jnp.float32)

        @pl.loop(0, RPS // EPB)
        def _(i):
            pltpu.sync_copy(rows[0], acc.at[pl.ds(sid * RPS + i * EPB, EPB)])

        plsc.subcore_barrier()
        wait_idx(0, 0)
        issue_g(0, 0)

        @pl.loop(0, nb // 4)
        def _(g):
            for j in range(4):
                b = g * 4 + j
                jr, jr1 = j % 2, (j + 1) % 2
                ji, ji1, ji3 = j, (j + 1) % 4, (j + 3) % 4

                @pl.when(b >= 1)
                def _():
                    wait_s(jr1, ji3)

                    @pl.when(b + 3 < nb)
                    def _():
                        issue_idx(b + 3, ji3)

                @pl.when(b + 1 < nb)
                def _():
                    wait_idx(b + 1, ji1)
                    issue_g(jr1, ji1)

                wait_g(jr, ji)
                issue_s(jr, ji)

        wait_s(1, 3)
        plsc.subcore_barrier()

        @pl.loop(0, RPS // EPB)
        def _(i):
            off = sid * RPS + i * EPB
            pltpu.sync_copy(acc.at[pl.ds(off, EPB)],
                            out_hbm.at[cid, pl.ds(off, EPB)])

    return k(table, sp, dp)


def _tc_pre(accd, x):
    def body(accd_ref, x_ref, xs_ref, d_ref):
        deg = accd_ref[0, :, 0:1] + accd_ref[1, :, 0:1] - 1.0
        d = lax.rsqrt(deg)
        d_ref[...] = d
        xs_ref[...] = x_ref[...] * d

    return pl.pallas_call(
        body,
        grid=(N // BR,),
        in_specs=[
            pl.BlockSpec((NC, BR, DEGW), lambda i: (0, i, 0)),
            pl.BlockSpec((BR, D), lambda i: (i, 0)),
        ],
        out_specs=[
            pl.BlockSpec((BR, D), lambda i: (i, 0)),
            pl.BlockSpec((BR, 1), lambda i: (i, 0)),
        ],
        out_shape=[
            jax.ShapeDtypeStruct((N, D), jnp.float32),
            jax.ShapeDtypeStruct((N, 1), jnp.float32),
        ],
    )(accd, x)


def _tc_mid(acc, xs, d, W1, b1):
    def body(acc_ref, xs_ref, d_ref, w_ref, b_ref, emb_ref, es_ref):
        d = d_ref[...]
        h = (acc_ref[0] + acc_ref[1] + xs_ref[...]) * d
        e = lax.dot_general(h, w_ref[...],
                            dimension_numbers=(((1,), (1,)), ((), ())),
                            preferred_element_type=jnp.float32)
        e = jnp.maximum(e + b_ref[...], 0.0)
        emb_ref[...] = e
        es_ref[...] = e * d

    return pl.pallas_call(
        body,
        grid=(N // BR,),
        in_specs=[
            pl.BlockSpec((NC, BR, D), lambda i: (0, i, 0)),
            pl.BlockSpec((BR, D), lambda i: (i, 0)),
            pl.BlockSpec((BR, 1), lambda i: (i, 0)),
            pl.BlockSpec((D, D), lambda i: (0, 0)),
            pl.BlockSpec((1, D), lambda i: (0, 0)),
        ],
        out_specs=[
            pl.BlockSpec((BR, D), lambda i: (i, 0)),
            pl.BlockSpec((BR, D), lambda i: (i, 0)),
        ],
        out_shape=[
            jax.ShapeDtypeStruct((N, D), jnp.float32),
            jax.ShapeDtypeStruct((N, D), jnp.float32),
        ],
    )(acc, xs, d, W1, b1)


def _tc_post(acc, es, d, W2, b2):
    def body(acc_ref, es_ref, d_ref, w_ref, b_ref, out_ref):
        h = (acc_ref[0] + acc_ref[1] + es_ref[...]) * d_ref[...]
        o = lax.dot_general(h, w_ref[...],
                            dimension_numbers=(((1,), (1,)), ((), ())),
                            preferred_element_type=jnp.float32)
        out_ref[...] = o + b_ref[...]

    return pl.pallas_call(
        body,
        grid=(N // BR,),
        in_specs=[
            pl.BlockSpec((NC, BR, D), lambda i: (0, i, 0)),
            pl.BlockSpec((BR, D), lambda i: (i, 0)),
            pl.BlockSpec((BR, 1), lambda i: (i, 0)),
            pl.BlockSpec((D, D), lambda i: (0, 0)),
            pl.BlockSpec((1, D), lambda i: (0, 0)),
        ],
        out_specs=pl.BlockSpec((BR, D), lambda i: (i, 0)),
        out_shape=jax.ShapeDtypeStruct((N, D), jnp.float32),
    )(acc, es, d, W2, b2)


def kernel(x, edge_index, W1, b1, W2, b2):
    sp = jnp.concatenate([edge_index[0], jnp.zeros((PAD,), jnp.int32)])
    trash = TRASH + jnp.arange(PAD, dtype=jnp.int32) % (ACC_ROWS - N)
    dp = jnp.concatenate([edge_index[1], trash])
    spd = sp.reshape(NW, NB_PER_W, EPB)
    dpd = dp.reshape(NW, NB_PER_W, EPB)
    spp = sp.reshape(NBT, EPB)
    dpp = dp.reshape(NBT, EPB)
    accd = _sc_degree(dpd)
    xs, d = _tc_pre(accd, x)
    acc1 = _sc_propagate(xs, spp, dpp)
    emb, es = _tc_mid(acc1, xs, d, W1, b1.reshape(1, D))
    acc2 = _sc_propagate(es, spp, dpp)
    out = _tc_post(acc2, es, d, W2, b2.reshape(1, D))
    return emb, out


# speedup vs baseline: 1.1157x; 1.0246x over previous
import functools

import jax
import jax.numpy as jnp
from jax import lax
from jax.experimental import pallas as pl
from jax.experimental.pallas import tpu as pltpu
from jax.experimental.pallas import tpu_sc as plsc

N = 10000
D = 128
E = 320000

NC = 2
NS = 16
NW = NC * NS
EPB = 128
NBUF = 4
NB_PER_W = 80
E_PAD = NB_PER_W * NW * EPB
PAD = E_PAD - E
ACC_ROWS = 10240
TRASH = N
RPS = ACC_ROWS // NS
ZROWS = 64
DEGW = 128
NBT = E_PAD // EPB
NB0 = 124
NB1 = NBT // NS - NB0

BR = 1000

_MESH = dict(core_axis_name="c", subcore_axis_name="s")


def _sc_degree(dp):
    @functools.partial(
        pl.kernel,
        out_type=jax.ShapeDtypeStruct((NC, ACC_ROWS, DEGW), jnp.float32),
        mesh=plsc.VectorSubcoreMesh(**_MESH),
        scratch_types=[pltpu.VMEM((EPB, DEGW), jnp.float32)]
        + [pltpu.VMEM((EPB,), jnp.int32)] * 4
        + [pltpu.VMEM_SHARED((ACC_ROWS, DEGW), jnp.float32)]
        + [pltpu.SemaphoreType.DMA] * 8,
    )
    def k(dp_hbm, out_hbm, *bufs):
        ones = bufs[0]
        didx = bufs[1:5]
        acc = bufs[5]
        isem, ssem = bufs[6:10], bufs[10:14]
        cid = lax.axis_index("c")
        sid = lax.axis_index("s")
        gwid = cid * NS + sid

        def issue_idx(b, ji):
            pltpu.async_copy(dp_hbm.at[gwid, b], didx[ji], isem[ji])

        def wait_idx(b, ji):
            pltpu.make_async_copy(dp_hbm.at[gwid, b], didx[ji],
                                  isem[ji]).wait()

        def issue_s(ji):
            pltpu.async_copy(ones, acc.at[didx[ji]], ssem[ji], add=True)

        def wait_s(ji):
            pltpu.make_async_copy(ones, acc.at[didx[ji]], ssem[ji]).wait()

        issue_idx(0, 0)

        @pl.loop(0, EPB)
        def _(r):
            @pl.loop(0, DEGW // 16)
            def _(c):
                ones[r, pl.ds(c * 16, 16)] = jnp.ones((16,), jnp.float32)

        @pl.loop(0, RPS // ZROWS)
        def _(i):
            pltpu.sync_copy(ones.at[pl.ds(0, ZROWS)],
                            acc.at[pl.ds(sid * RPS + i * ZROWS, ZROWS)])

        plsc.subcore_barrier()

        @pl.loop(0, NB_PER_W // 4)
        def _(g):
            for j in range(4):
                b = g * 4 + j
                ji1, ji3 = (j + 1) % 4, (j + 3) % 4

                @pl.when(b >= 3)
                def _():
                    wait_s(ji1)

                @pl.when(b + 1 < NB_PER_W)
                def _():
                    issue_idx(b + 1, ji1)

                wait_idx(b, j)
                issue_s(j)

        for j in (1, 2, 3):
            wait_s(j)

        plsc.subcore_barrier()

        @pl.loop(0, RPS // ZROWS)
        def _(i):
            off = sid * RPS + i * ZROWS
            pltpu.sync_copy(acc.at[pl.ds(off, ZROWS)],
                            out_hbm.at[cid, pl.ds(off, ZROWS)])

    return k(dp)


def _sc_propagate(table, sp, dp):
    @functools.partial(
        pl.kernel,
        out_type=jax.ShapeDtypeStruct((NC, ACC_ROWS, D), jnp.float32),
        mesh=plsc.VectorSubcoreMesh(**_MESH),
        scratch_types=[pltpu.VMEM((EPB, D), jnp.float32)] * 2
        + [pltpu.VMEM((EPB,), jnp.int32)] * 8
        + [pltpu.VMEM_SHARED((ACC_ROWS, D), jnp.float32)]
        + [pltpu.SemaphoreType.DMA] * 8,
    )
    def k(tab_hbm, sp_hbm, dp_hbm, out_hbm, *bufs):
        rows = bufs[0:2]
        sidx = bufs[2:6]
        didx = bufs[6:10]
        acc = bufs[10]
        gsem, ssem = bufs[11:13], bufs[13:15]
        isem = bufs[15:19]
        cid = lax.axis_index("c")
        sid = lax.axis_index("s")
        nb = jnp.where(cid == 0, NB0, NB1)
        base = jnp.where(cid == 0, sid * NB0, NS * NB0 + sid * NB1)

        def issue_idx(b, ji):
            pltpu.async_copy(sp_hbm.at[base + b], sidx[ji], isem[ji])
            pltpu.async_copy(dp_hbm.at[base + b], didx[ji], isem[ji])

        def wait_idx(b, ji):
            pltpu.make_async_copy(sp_hbm.at[base + b], sidx[ji],
                                  isem[ji]).wait()
            pltpu.make_async_copy(dp_hbm.at[base + b], didx[ji],
                                  isem[ji]).wait()

        def issue_g(jr, ji):
            pltpu.async_copy(tab_hbm.at[sidx[ji]], rows[jr], gsem[jr])

        def wait_g(jr, ji):
            pltpu.make_async_copy(tab_hbm.at[sidx[ji]], rows[jr],
                                  gsem[jr]).wait()

        def issue_s(jr, ji):
            pltpu.async_copy(rows[jr], acc.at[didx[ji]], ssem[jr], add=True)

        def wait_s(jr, ji):
            pltpu.make_async_copy(rows[jr], acc.at[didx[ji]],
                                  ssem[jr]).wait()

        for j in range(4):
            issue_idx(j, j)

        @pl.loop(0, EPB)
        def _(r):
            @pl.loop(0, D // 16)
            def _(c):
                rows[0][r, pl.ds(c * 16, 16)] = jnp.zeros((16,), jnp.float32)

        @pl.loop(0, RPS // EPB)
        def _(i):
            pltpu.sync_copy(rows[0], acc.at[pl.ds(sid * RPS + i * EPB, EPB)])

        plsc.subcore_barrier()
        wait_idx(0, 0)
        issue_g(0, 0)

        @pl.loop(0, nb // 4)
        def _(g):
            for j in range(4):
                b = g * 4 + j
                jr, jr1 = j % 2, (j + 1) % 2
                ji, ji1, ji3 = j, (j + 1) % 4, (j + 3) % 4

                @pl.when(b >= 1)
                def _():
                    wait_s(jr1, ji3)

                    @pl.when(b + 3 < nb)
                    def _():
                        issue_idx(b + 3, ji3)

                @pl.when(b + 1 < nb)
                def _():
                    wait_idx(b + 1, ji1)
                    issue_g(jr1, ji1)

                wait_g(jr, ji)
                issue_s(jr, ji)

        wait_s(1, 3)
        plsc.subcore_barrier()

        @pl.loop(0, RPS // EPB)
        def _(i):
            off = sid * RPS + i * EPB
            pltpu.sync_copy(acc.at[pl.ds(off, EPB)],
                            out_hbm.at[cid, pl.ds(off, EPB)])

    return k(table, sp, dp)


def _tc_pre(accd, x):
    def body(accd_ref, x_ref, xs_ref, d_ref):
        deg = accd_ref[0, :, 0:1] + accd_ref[1, :, 0:1] - 1.0
        d = lax.rsqrt(deg)
        d_ref[...] = d
        xs_ref[...] = x_ref[...] * d

    return pl.pallas_call(
        body,
        grid=(N // BR,),
        in_specs=[
            pl.BlockSpec((NC, BR, DEGW), lambda i: (0, i, 0)),
            pl.BlockSpec((BR, D), lambda i: (i, 0)),
        ],
        out_specs=[
            pl.BlockSpec((BR, D), lambda i: (i, 0)),
            pl.BlockSpec((BR, 1), lambda i: (i, 0)),
        ],
        out_shape=[
            jax.ShapeDtypeStruct((N, D), jnp.float32),
            jax.ShapeDtypeStruct((N, 1), jnp.float32),
        ],
    )(accd, x)


def _tc_mid(acc, xs, d, W1, b1):
    def body(acc_ref, xs_ref, d_ref, w_ref, b_ref, emb_ref, es_ref):
        d = d_ref[...]
        h = (acc_ref[0] + acc_ref[1] + xs_ref[...]) * d
        e = lax.dot_general(h, w_ref[...],
                            dimension_numbers=(((1,), (1,)), ((), ())),
                            preferred_element_type=jnp.float32)
        e = jnp.maximum(e + b_ref[...], 0.0)
        emb_ref[...] = e
        es_ref[...] = e * d

    return pl.pallas_call(
        body,
        grid=(N // BR,),
        in_specs=[
            pl.BlockSpec((NC, BR, D), lambda i: (0, i, 0)),
            pl.BlockSpec((BR, D), lambda i: (i, 0)),
            pl.BlockSpec((BR, 1), lambda i: (i, 0)),
            pl.BlockSpec((D, D), lambda i: (0, 0)),
            pl.BlockSpec((1, D), lambda i: (0, 0)),
        ],
        out_specs=[
            pl.BlockSpec((BR, D), lambda i: (i, 0)),
            pl.BlockSpec((BR, D), lambda i: (i, 0)),
        ],
        out_shape=[
            jax.ShapeDtypeStruct((N, D), jnp.float32),
            jax.ShapeDtypeStruct((N, D), jnp.float32),
        ],
    )(acc, xs, d, W1, b1)


def _tc_post(acc, es, d, W2, b2):
    def body(acc_ref, es_ref, d_ref, w_ref, b_ref, out_ref):
        h = (acc_ref[0] + acc_ref[1] + es_ref[...]) * d_ref[...]
        o = lax.dot_general(h, w_ref[...],
                            dimension_numbers=(((1,), (1,)), ((), ())),
                            preferred_element_type=jnp.float32)
        out_ref[...] = o + b_ref[...]

    return pl.pallas_call(
        body,
        grid=(N // BR,),
        in_specs=[
            pl.BlockSpec((NC, BR, D), lambda i: (0, i, 0)),
            pl.BlockSpec((BR, D), lambda i: (i, 0)),
            pl.BlockSpec((BR, 1), lambda i: (i, 0)),
            pl.BlockSpec((D, D), lambda i: (0, 0)),
            pl.BlockSpec((1, D), lambda i: (0, 0)),
        ],
        out_specs=pl.BlockSpec((BR, D), lambda i: (i, 0)),
        out_shape=jax.ShapeDtypeStruct((N, D), jnp.float32),
    )(acc, es, d, W2, b2)


def kernel(x, edge_index, W1, b1, W2, b2):
    sp = jnp.concatenate([edge_index[0], jnp.zeros((PAD,), jnp.int32)])
    trash = TRASH + jnp.arange(PAD, dtype=jnp.int32) % (ACC_ROWS - N)
    dp = jnp.concatenate([edge_index[1], trash])
    spd = sp.reshape(NW, NB_PER_W, EPB)
    dpd = dp.reshape(NW, NB_PER_W, EPB)
    spp = sp.reshape(NBT, EPB)
    dpp = dp.reshape(NBT, EPB)
    accd = _sc_degree(dpd)
    xs, d = _tc_pre(accd, x)
    acc1 = _sc_propagate(xs, spp, dpp)
    emb, es = _tc_mid(acc1, xs, d, W1, b1.reshape(1, D))
    acc2 = _sc_propagate(es, spp, dpp)
    out = _tc_post(acc2, es, d, W2, b2.reshape(1, D))
    return emb, out


# speedup vs baseline: 1.1271x; 1.0103x over previous
import functools

import jax
import jax.numpy as jnp
from jax import lax
from jax.experimental import pallas as pl
from jax.experimental.pallas import tpu as pltpu
from jax.experimental.pallas import tpu_sc as plsc

N = 10000
D = 128
E = 320000

NC = 2
NS = 16
NW = NC * NS
EPB = 128
NBUF = 4
NB_PER_W = 80
E_PAD = NB_PER_W * NW * EPB
PAD = E_PAD - E
ACC_ROWS = 10240
TRASH = N
RPS = ACC_ROWS // NS
ZROWS = 64
DEGW = 128
NBT = E_PAD // EPB
NB0 = 124
NB1 = NBT // NS - NB0

BR = 1000

_MESH = dict(core_axis_name="c", subcore_axis_name="s")


def _sc_degree(dp):
    @functools.partial(
        pl.kernel,
        out_type=jax.ShapeDtypeStruct((NC, ACC_ROWS, DEGW), jnp.float32),
        mesh=plsc.VectorSubcoreMesh(**_MESH),
        scratch_types=[pltpu.VMEM((EPB, DEGW), jnp.float32)]
        + [pltpu.VMEM((EPB,), jnp.int32)] * 4
        + [pltpu.VMEM_SHARED((ACC_ROWS, DEGW), jnp.float32)]
        + [pltpu.SemaphoreType.DMA] * 8,
    )
    def k(dp_hbm, out_hbm, *bufs):
        ones = bufs[0]
        didx = bufs[1:5]
        acc = bufs[5]
        isem, ssem = bufs[6:10], bufs[10:14]
        cid = lax.axis_index("c")
        sid = lax.axis_index("s")
        gwid = cid * NS + sid

        def issue_idx(b, ji):
            pltpu.async_copy(dp_hbm.at[gwid, b], didx[ji], isem[ji])

        def wait_idx(b, ji):
            pltpu.make_async_copy(dp_hbm.at[gwid, b], didx[ji],
                                  isem[ji]).wait()

        def issue_s(ji):
            pltpu.async_copy(ones, acc.at[didx[ji]], ssem[ji], add=True)

        def wait_s(ji):
            pltpu.make_async_copy(ones, acc.at[didx[ji]], ssem[ji]).wait()

        issue_idx(0, 0)

        @pl.loop(0, EPB)
        def _(r):
            @pl.loop(0, DEGW // 16)
            def _(c):
                ones[r, pl.ds(c * 16, 16)] = jnp.ones((16,), jnp.float32)

        @pl.loop(0, RPS // ZROWS)
        def _(i):
            pltpu.sync_copy(ones.at[pl.ds(0, ZROWS)],
                            acc.at[pl.ds(sid * RPS + i * ZROWS, ZROWS)])

        plsc.subcore_barrier()

        @pl.loop(0, NB_PER_W // 4)
        def _(g):
            for j in range(4):
                b = g * 4 + j
                ji1, ji3 = (j + 1) % 4, (j + 3) % 4

                @pl.when(b >= 3)
                def _():
                    wait_s(ji1)

                @pl.when(b + 1 < NB_PER_W)
                def _():
                    issue_idx(b + 1, ji1)

                wait_idx(b, j)
                issue_s(j)

        for j in (1, 2, 3):
            wait_s(j)

        plsc.subcore_barrier()

        @pl.loop(0, RPS // ZROWS)
        def _(i):
            off = sid * RPS + i * ZROWS
            pltpu.sync_copy(acc.at[pl.ds(off, ZROWS)],
                            out_hbm.at[cid, pl.ds(off, ZROWS)])

    return k(dp)


def _sc_propagate(table, sp, dp):
    @functools.partial(
        pl.kernel,
        out_type=jax.ShapeDtypeStruct((NC, ACC_ROWS, D), jnp.float32),
        mesh=plsc.VectorSubcoreMesh(**_MESH),
        scratch_types=[pltpu.VMEM((EPB, D), jnp.float32)] * 2
        + [pltpu.VMEM((EPB // 2,), jnp.int32)] * 8
        + [pltpu.VMEM((EPB,), jnp.int32)] * 4
        + [pltpu.VMEM_SHARED((ACC_ROWS, D), jnp.float32)]
        + [pltpu.SemaphoreType.DMA] * 8,
    )
    def k(tab_hbm, sp_hbm, dp_hbm, out_hbm, *bufs):
        rows = bufs[0:2]
        sidx = bufs[2:10]
        didx = bufs[10:14]
        acc = bufs[14]
        gsem, ssem = bufs[15:17], bufs[17:19]
        isem = bufs[19:23]
        cid = lax.axis_index("c")
        sid = lax.axis_index("s")
        nb = jnp.where(cid == 0, NB0, NB1)
        base = jnp.where(cid == 0, sid * NB0, NS * NB0 + sid * NB1)
        H = EPB // 2

        def issue_idx(b, ji):
            pltpu.async_copy(sp_hbm.at[base + b, pl.ds(0, H)],
                             sidx[2 * ji], isem[ji])
            pltpu.async_copy(sp_hbm.at[base + b, pl.ds(H, H)],
                             sidx[2 * ji + 1], isem[ji])
            pltpu.async_copy(dp_hbm.at[base + b], didx[ji], isem[ji])

        def wait_idx(b, ji):
            pltpu.make_async_copy(sp_hbm.at[base + b, pl.ds(0, H)],
                                  sidx[2 * ji], isem[ji]).wait()
            pltpu.make_async_copy(sp_hbm.at[base + b, pl.ds(H, H)],
                                  sidx[2 * ji + 1], isem[ji]).wait()
            pltpu.make_async_copy(dp_hbm.at[base + b], didx[ji],
                                  isem[ji]).wait()

        def issue_g(jr, ji):
            pltpu.async_copy(tab_hbm.at[sidx[2 * ji]],
                             rows[jr].at[pl.ds(0, H)], gsem[jr])
            pltpu.async_copy(tab_hbm.at[sidx[2 * ji + 1]],
                             rows[jr].at[pl.ds(H, H)], gsem[jr])

        def wait_g(jr, ji):
            pltpu.make_async_copy(tab_hbm.at[sidx[2 * ji]],
                                  rows[jr].at[pl.ds(0, H)], gsem[jr]).wait()
            pltpu.make_async_copy(tab_hbm.at[sidx[2 * ji + 1]],
                                  rows[jr].at[pl.ds(H, H)], gsem[jr]).wait()

        def issue_s(jr, ji):
            pltpu.async_copy(rows[jr], acc.at[didx[ji]], ssem[jr], add=True)

        def wait_s(jr, ji):
            pltpu.make_async_copy(rows[jr], acc.at[didx[ji]],
                                  ssem[jr]).wait()

        for j in range(4):
            issue_idx(j, j)

        @pl.loop(0, EPB)
        def _(r):
            @pl.loop(0, D // 16)
            def _(c):
                rows[0][r, pl.ds(c * 16, 16)] = jnp.zeros((16,), jnp.float32)

        @pl.loop(0, RPS // EPB)
        def _(i):
            pltpu.sync_copy(rows[0], acc.at[pl.ds(sid * RPS + i * EPB, EPB)])

        plsc.subcore_barrier()
        wait_idx(0, 0)
        issue_g(0, 0)

        @pl.loop(0, nb // 4)
        def _(g):
            for j in range(4):
                b = g * 4 + j
                jr, jr1 = j % 2, (j + 1) % 2
                ji, ji1, ji3 = j, (j + 1) % 4, (j + 3) % 4

                @pl.when(b >= 1)
                def _():
                    wait_s(jr1, ji3)

                    @pl.when(b + 3 < nb)
                    def _():
                        issue_idx(b + 3, ji3)

                @pl.when(b + 1 < nb)
                def _():
                    wait_idx(b + 1, ji1)
                    issue_g(jr1, ji1)

                wait_g(jr, ji)
                issue_s(jr, ji)

        wait_s(1, 3)
        plsc.subcore_barrier()

        @pl.loop(0, RPS // EPB)
        def _(i):
            off = sid * RPS + i * EPB
            pltpu.sync_copy(acc.at[pl.ds(off, EPB)],
                            out_hbm.at[cid, pl.ds(off, EPB)])

    return k(table, sp, dp)


def _tc_pre(accd, x):
    def body(accd_ref, x_ref, xs_ref, d_ref):
        deg = accd_ref[0, :, 0:1] + accd_ref[1, :, 0:1] - 1.0
        d = lax.rsqrt(deg)
        d_ref[...] = d
        xs_ref[...] = x_ref[...] * d

    return pl.pallas_call(
        body,
        grid=(N // BR,),
        in_specs=[
            pl.BlockSpec((NC, BR, DEGW), lambda i: (0, i, 0)),
            pl.BlockSpec((BR, D), lambda i: (i, 0)),
        ],
        out_specs=[
            pl.BlockSpec((BR, D), lambda i: (i, 0)),
            pl.BlockSpec((BR, 1), lambda i: (i, 0)),
        ],
        out_shape=[
            jax.ShapeDtypeStruct((N, D), jnp.float32),
            jax.ShapeDtypeStruct((N, 1), jnp.float32),
        ],
    )(accd, x)


def _tc_mid(acc, xs, d, W1, b1):
    def body(acc_ref, xs_ref, d_ref, w_ref, b_ref, emb_ref, es_ref):
        d = d_ref[...]
        h = (acc_ref[0] + acc_ref[1] + xs_ref[...]) * d
        e = lax.dot_general(h, w_ref[...],
                            dimension_numbers=(((1,), (1,)), ((), ())),
                            preferred_element_type=jnp.float32)
        e = jnp.maximum(e + b_ref[...], 0.0)
        emb_ref[...] = e
        es_ref[...] = e * d

    return pl.pallas_call(
        body,
        grid=(N // BR,),
        in_specs=[
            pl.BlockSpec((NC, BR, D), lambda i: (0, i, 0)),
            pl.BlockSpec((BR, D), lambda i: (i, 0)),
            pl.BlockSpec((BR, 1), lambda i: (i, 0)),
            pl.BlockSpec((D, D), lambda i: (0, 0)),
            pl.BlockSpec((1, D), lambda i: (0, 0)),
        ],
        out_specs=[
            pl.BlockSpec((BR, D), lambda i: (i, 0)),
            pl.BlockSpec((BR, D), lambda i: (i, 0)),
        ],
        out_shape=[
            jax.ShapeDtypeStruct((N, D), jnp.float32),
            jax.ShapeDtypeStruct((N, D), jnp.float32),
        ],
    )(acc, xs, d, W1, b1)


def _tc_post(acc, es, d, W2, b2):
    def body(acc_ref, es_ref, d_ref, w_ref, b_ref, out_ref):
        h = (acc_ref[0] + acc_ref[1] + es_ref[...]) * d_ref[...]
        o = lax.dot_general(h, w_ref[...],
                            dimension_numbers=(((1,), (1,)), ((), ())),
                            preferred_element_type=jnp.float32)
        out_ref[...] = o + b_ref[...]

    return pl.pallas_call(
        body,
        grid=(N // BR,),
        in_specs=[
            pl.BlockSpec((NC, BR, D), lambda i: (0, i, 0)),
            pl.BlockSpec((BR, D), lambda i: (i, 0)),
            pl.BlockSpec((BR, 1), lambda i: (i, 0)),
            pl.BlockSpec((D, D), lambda i: (0, 0)),
            pl.BlockSpec((1, D), lambda i: (0, 0)),
        ],
        out_specs=pl.BlockSpec((BR, D), lambda i: (i, 0)),
        out_shape=jax.ShapeDtypeStruct((N, D), jnp.float32),
    )(acc, es, d, W2, b2)


def kernel(x, edge_index, W1, b1, W2, b2):
    sp = jnp.concatenate([edge_index[0], jnp.zeros((PAD,), jnp.int32)])
    trash = TRASH + jnp.arange(PAD, dtype=jnp.int32) % (ACC_ROWS - N)
    dp = jnp.concatenate([edge_index[1], trash])
    spd = sp.reshape(NW, NB_PER_W, EPB)
    dpd = dp.reshape(NW, NB_PER_W, EPB)
    spp = sp.reshape(NBT, EPB)
    dpp = dp.reshape(NBT, EPB)
    accd = _sc_degree(dpd)
    xs, d = _tc_pre(accd, x)
    acc1 = _sc_propagate(xs, spp, dpp)
    emb, es = _tc_mid(acc1, xs, d, W1, b1.reshape(1, D))
    acc2 = _sc_propagate(es, spp, dpp)
    out = _tc_post(acc2, es, d, W2, b2.reshape(1, D))
    return emb, out


# speedup vs baseline: 1.1973x; 1.0622x over previous
import functools

import jax
import jax.numpy as jnp
from jax import lax
from jax.experimental import pallas as pl
from jax.experimental.pallas import tpu as pltpu
from jax.experimental.pallas import tpu_sc as plsc

N = 10000
D = 128
E = 320000

NC = 2
NS = 16
NW = NC * NS
EPB = 128
NBUF = 4
NB_PER_W = 80
E_PAD = NB_PER_W * NW * EPB
PAD = E_PAD - E
ACC_ROWS = 10240
TRASH = N
RPS = ACC_ROWS // NS
ZROWS = 64
DEGW = 128
NBT = E_PAD // EPB
NB0 = 156
NB1 = NBT // NS - NB0

BR = 1000

_MESH = dict(core_axis_name="c", subcore_axis_name="s")


def _sc_degree(dp):
    @functools.partial(
        pl.kernel,
        out_type=jax.ShapeDtypeStruct((NC, ACC_ROWS, DEGW), jnp.float32),
        mesh=plsc.VectorSubcoreMesh(**_MESH),
        scratch_types=[pltpu.VMEM((EPB, DEGW), jnp.float32)]
        + [pltpu.VMEM((EPB,), jnp.int32)] * 4
        + [pltpu.VMEM_SHARED((ACC_ROWS, DEGW), jnp.float32)]
        + [pltpu.SemaphoreType.DMA] * 8,
    )
    def k(dp_hbm, out_hbm, *bufs):
        ones = bufs[0]
        didx = bufs[1:5]
        acc = bufs[5]
        isem, ssem = bufs[6:10], bufs[10:14]
        cid = lax.axis_index("c")
        sid = lax.axis_index("s")
        gwid = cid * NS + sid

        def issue_idx(b, ji):
            pltpu.async_copy(dp_hbm.at[gwid, b], didx[ji], isem[ji])

        def wait_idx(b, ji):
            pltpu.make_async_copy(dp_hbm.at[gwid, b], didx[ji],
                                  isem[ji]).wait()

        def issue_s(ji):
            pltpu.async_copy(ones, acc.at[didx[ji]], ssem[ji], add=True)

        def wait_s(ji):
            pltpu.make_async_copy(ones, acc.at[didx[ji]], ssem[ji]).wait()

        issue_idx(0, 0)

        @pl.loop(0, EPB)
        def _(r):
            @pl.loop(0, DEGW // 16)
            def _(c):
                ones[r, pl.ds(c * 16, 16)] = jnp.ones((16,), jnp.float32)

        @pl.loop(0, RPS // ZROWS)
        def _(i):
            pltpu.sync_copy(ones.at[pl.ds(0, ZROWS)],
                            acc.at[pl.ds(sid * RPS + i * ZROWS, ZROWS)])

        plsc.subcore_barrier()

        @pl.loop(0, NB_PER_W // 4)
        def _(g):
            for j in range(4):
                b = g * 4 + j
                ji1, ji3 = (j + 1) % 4, (j + 3) % 4

                @pl.when(b >= 3)
                def _():
                    wait_s(ji1)

                @pl.when(b + 1 < NB_PER_W)
                def _():
                    issue_idx(b + 1, ji1)

                wait_idx(b, j)
                issue_s(j)

        for j in (1, 2, 3):
            wait_s(j)

        plsc.subcore_barrier()

        @pl.loop(0, RPS // ZROWS)
        def _(i):
            off = sid * RPS + i * ZROWS
            pltpu.sync_copy(acc.at[pl.ds(off, ZROWS)],
                            out_hbm.at[cid, pl.ds(off, ZROWS)])

    return k(dp)


def _sc_propagate(table, sp, dp):
    @functools.partial(
        pl.kernel,
        out_type=jax.ShapeDtypeStruct((NC, ACC_ROWS, D), jnp.float32),
        mesh=plsc.VectorSubcoreMesh(**_MESH),
        scratch_types=[pltpu.VMEM((EPB, D), jnp.float32)] * 2
        + [pltpu.VMEM((EPB // 2,), jnp.int32)] * 8
        + [pltpu.VMEM((EPB,), jnp.int32)] * 4
        + [pltpu.VMEM_SHARED((ACC_ROWS, D), jnp.float32)]
        + [pltpu.SemaphoreType.DMA] * 8,
    )
    def k(tab_hbm, sp_hbm, dp_hbm, out_hbm, *bufs):
        rows = bufs[0:2]
        sidx = bufs[2:10]
        didx = bufs[10:14]
        acc = bufs[14]
        gsem, ssem = bufs[15:17], bufs[17:19]
        isem = bufs[19:23]
        cid = lax.axis_index("c")
        sid = lax.axis_index("s")
        nb = jnp.where(cid == 0, NB0, NB1)
        base = jnp.where(cid == 0, sid * NB0, NS * NB0 + sid * NB1)
        H = EPB // 2

        def issue_idx(b, ji):
            pltpu.async_copy(sp_hbm.at[base + b, pl.ds(0, H)],
                             sidx[2 * ji], isem[ji])
            pltpu.async_copy(sp_hbm.at[base + b, pl.ds(H, H)],
                             sidx[2 * ji + 1], isem[ji])
            pltpu.async_copy(dp_hbm.at[base + b], didx[ji], isem[ji])

        def wait_idx(b, ji):
            pltpu.make_async_copy(sp_hbm.at[base + b, pl.ds(0, H)],
                                  sidx[2 * ji], isem[ji]).wait()
            pltpu.make_async_copy(sp_hbm.at[base + b, pl.ds(H, H)],
                                  sidx[2 * ji + 1], isem[ji]).wait()
            pltpu.make_async_copy(dp_hbm.at[base + b], didx[ji],
                                  isem[ji]).wait()

        def issue_g(jr, ji):
            pltpu.async_copy(tab_hbm.at[sidx[2 * ji]],
                             rows[jr].at[pl.ds(0, H)], gsem[jr])
            pltpu.async_copy(tab_hbm.at[sidx[2 * ji + 1]],
                             rows[jr].at[pl.ds(H, H)], gsem[jr])

        def wait_g(jr, ji):
            pltpu.make_async_copy(tab_hbm.at[sidx[2 * ji]],
                                  rows[jr].at[pl.ds(0, H)], gsem[jr]).wait()
            pltpu.make_async_copy(tab_hbm.at[sidx[2 * ji + 1]],
                                  rows[jr].at[pl.ds(H, H)], gsem[jr]).wait()

        def issue_s(jr, ji):
            pltpu.async_copy(rows[jr], acc.at[didx[ji]], ssem[jr], add=True)

        def wait_s(jr, ji):
            pltpu.make_async_copy(rows[jr], acc.at[didx[ji]],
                                  ssem[jr]).wait()

        for j in range(4):
            issue_idx(j, j)

        @pl.loop(0, EPB)
        def _(r):
            @pl.loop(0, D // 16)
            def _(c):
                rows[0][r, pl.ds(c * 16, 16)] = jnp.zeros((16,), jnp.float32)

        @pl.loop(0, RPS // EPB)
        def _(i):
            pltpu.sync_copy(rows[0], acc.at[pl.ds(sid * RPS + i * EPB, EPB)])

        plsc.subcore_barrier()
        wait_idx(0, 0)
        issue_g(0, 0)

        @pl.loop(0, nb // 4)
        def _(g):
            for j in range(4):
                b = g * 4 + j
                jr, jr1 = j % 2, (j + 1) % 2
                ji, ji1, ji3 = j, (j + 1) % 4, (j + 3) % 4

                @pl.when(b >= 1)
                def _():
                    wait_s(jr1, ji3)

                    @pl.when(b + 3 < nb)
                    def _():
                        issue_idx(b + 3, ji3)

                @pl.when(b + 1 < nb)
                def _():
                    wait_idx(b + 1, ji1)
                    issue_g(jr1, ji1)

                wait_g(jr, ji)
                issue_s(jr, ji)

        wait_s(1, 3)
        plsc.subcore_barrier()

        @pl.loop(0, RPS // EPB)
        def _(i):
            off = sid * RPS + i * EPB
            pltpu.sync_copy(acc.at[pl.ds(off, EPB)],
                            out_hbm.at[cid, pl.ds(off, EPB)])

    return k(table, sp, dp)


def _tc_pre(accd, x):
    def body(accd_ref, x_ref, xs_ref, d_ref):
        deg = accd_ref[0, :, 0:1] + accd_ref[1, :, 0:1] - 1.0
        d = lax.rsqrt(deg)
        d_ref[...] = d
        xs_ref[...] = x_ref[...] * d

    return pl.pallas_call(
        body,
        grid=(N // BR,),
        in_specs=[
            pl.BlockSpec((NC, BR, DEGW), lambda i: (0, i, 0)),
            pl.BlockSpec((BR, D), lambda i: (i, 0)),
        ],
        out_specs=[
            pl.BlockSpec((BR, D), lambda i: (i, 0)),
            pl.BlockSpec((BR, 1), lambda i: (i, 0)),
        ],
        out_shape=[
            jax.ShapeDtypeStruct((N, D), jnp.float32),
            jax.ShapeDtypeStruct((N, 1), jnp.float32),
        ],
    )(accd, x)


def _tc_mid(acc, xs, d, W1, b1):
    def body(acc_ref, xs_ref, d_ref, w_ref, b_ref, emb_ref, es_ref):
        d = d_ref[...]
        h = (acc_ref[0] + acc_ref[1] + xs_ref[...]) * d
        e = lax.dot_general(h, w_ref[...],
                            dimension_numbers=(((1,), (1,)), ((), ())),
                            preferred_element_type=jnp.float32)
        e = jnp.maximum(e + b_ref[...], 0.0)
        emb_ref[...] = e
        es_ref[...] = e * d

    return pl.pallas_call(
        body,
        grid=(N // BR,),
        in_specs=[
            pl.BlockSpec((NC, BR, D), lambda i: (0, i, 0)),
            pl.BlockSpec((BR, D), lambda i: (i, 0)),
            pl.BlockSpec((BR, 1), lambda i: (i, 0)),
            pl.BlockSpec((D, D), lambda i: (0, 0)),
            pl.BlockSpec((1, D), lambda i: (0, 0)),
        ],
        out_specs=[
            pl.BlockSpec((BR, D), lambda i: (i, 0)),
            pl.BlockSpec((BR, D), lambda i: (i, 0)),
        ],
        out_shape=[
            jax.ShapeDtypeStruct((N, D), jnp.float32),
            jax.ShapeDtypeStruct((N, D), jnp.float32),
        ],
    )(acc, xs, d, W1, b1)


def _tc_post(acc, es, d, W2, b2):
    def body(acc_ref, es_ref, d_ref, w_ref, b_ref, out_ref):
        h = (acc_ref[0] + acc_ref[1] + es_ref[...]) * d_ref[...]
        o = lax.dot_general(h, w_ref[...],
                            dimension_numbers=(((1,), (1,)), ((), ())),
                            preferred_element_type=jnp.float32)
        out_ref[...] = o + b_ref[...]

    return pl.pallas_call(
        body,
        grid=(N // BR,),
        in_specs=[
            pl.BlockSpec((NC, BR, D), lambda i: (0, i, 0)),
            pl.BlockSpec((BR, D), lambda i: (i, 0)),
            pl.BlockSpec((BR, 1), lambda i: (i, 0)),
            pl.BlockSpec((D, D), lambda i: (0, 0)),
            pl.BlockSpec((1, D), lambda i: (0, 0)),
        ],
        out_specs=pl.BlockSpec((BR, D), lambda i: (i, 0)),
        out_shape=jax.ShapeDtypeStruct((N, D), jnp.float32),
    )(acc, es, d, W2, b2)


def kernel(x, edge_index, W1, b1, W2, b2):
    sp = jnp.concatenate([edge_index[0], jnp.zeros((PAD,), jnp.int32)])
    trash = TRASH + jnp.arange(PAD, dtype=jnp.int32) % (ACC_ROWS - N)
    dp = jnp.concatenate([edge_index[1], trash])
    spd = sp.reshape(NW, NB_PER_W, EPB)
    dpd = dp.reshape(NW, NB_PER_W, EPB)
    spp = sp.reshape(NBT, EPB)
    dpp = dp.reshape(NBT, EPB)
    accd = _sc_degree(dpd)
    xs, d = _tc_pre(accd, x)
    acc1 = _sc_propagate(xs, spp, dpp)
    emb, es = _tc_mid(acc1, xs, d, W1, b1.reshape(1, D))
    acc2 = _sc_propagate(es, spp, dpp)
    out = _tc_post(acc2, es, d, W2, b2.reshape(1, D))
    return emb, out
